# baseline (speedup 1.0000x reference)
import functools

import jax
import jax.numpy as jnp
from jax import lax
from jax.experimental import pallas as pl
from jax.experimental.pallas import tpu as pltpu
from jax.experimental.pallas import tpu_sc as plsc

NUM_ACTIONS = 100
ACTION_DIM = 1000
PATCHDES_DIM = 256
EMB_DIM = 128
BATCH = 1024
NUM_PATCHES = 196
ACT_VOCAB = NUM_ACTIONS * ACTION_DIM
SEQ = NUM_PATCHES + NUM_ACTIONS

NSPLIT = 4
BH = BATCH // NSPLIT
ROWS_H = BH * NUM_ACTIONS
NW = 32
PER_W = ROWS_H // NW
CHUNK = 80
NCHUNK = PER_W // CHUNK

BSZ = 64


def _sc_body(idx_hbm, table_hbm, out_hbm, idx_v, rows_v, sem):
    wid = lax.axis_index("s") * 2 + lax.axis_index("c")
    base = wid * PER_W
    pltpu.sync_copy(idx_hbm.at[pl.ds(base, PER_W)], idx_v)

    def step(j, carry):
        off = pl.multiple_of(j * CHUNK, CHUNK)
        pltpu.async_copy(
            table_hbm.at[idx_v.at[pl.ds(off, CHUNK)]], rows_v, sem
        ).wait()
        pltpu.sync_copy(rows_v, out_hbm.at[pl.ds(base + off, CHUNK)])
        return carry

    lax.fori_loop(0, NCHUNK, step, 0)


@functools.lru_cache(maxsize=1)
def _sc_gather():
    return pl.kernel(
        _sc_body,
        out_type=jax.ShapeDtypeStruct((ROWS_H, EMB_DIM), jnp.float32),
        mesh=plsc.VectorSubcoreMesh(core_axis_name="c", subcore_axis_name="s"),
        scratch_types=[
            pltpu.VMEM((PER_W,), jnp.int32),
            pltpu.VMEM((CHUNK, EMB_DIM), jnp.float32),
            pltpu.SemaphoreType.DMA,
        ],
    )


def _tc_body_first(p_ref, w_ref, b_ref, a_ref, o_ref):
    x = p_ref[...].reshape(-1, PATCHDES_DIM)
    y = jnp.dot(x, w_ref[...], preferred_element_type=jnp.float32) + b_ref[...]
    o_ref[:, :NUM_PATCHES, :] = y.reshape(-1, NUM_PATCHES, EMB_DIM)
    o_ref[:, NUM_PATCHES:, :] = a_ref[...]


def _tc_body_chain(prev_ref, p_ref, w_ref, b_ref, a_ref, o_ref):
    del prev_ref
    _tc_body_first(p_ref, w_ref, b_ref, a_ref, o_ref)


def _tc_call(s, prev_out, patches, W_obs, b_obs, act_s):
    grid = BH // BSZ
    common = dict(
        grid=(grid,),
        out_specs=pl.BlockSpec(
            (BSZ, SEQ, EMB_DIM), lambda i, s=s: (i + s * (BH // BSZ), 0, 0)
        ),
        out_shape=jax.ShapeDtypeStruct((BATCH, SEQ, EMB_DIM), jnp.float32),
        compiler_params=pltpu.CompilerParams(
            dimension_semantics=("arbitrary",),
        ),
    )
    data_specs = [
        pl.BlockSpec(
            (BSZ, NUM_PATCHES, PATCHDES_DIM),
            lambda i, s=s: (i + s * (BH // BSZ), 0, 0),
        ),
        pl.BlockSpec((PATCHDES_DIM, EMB_DIM), lambda i: (0, 0)),
        pl.BlockSpec((1, EMB_DIM), lambda i: (0, 0)),
        pl.BlockSpec((BSZ, NUM_ACTIONS, EMB_DIM), lambda i: (i, 0, 0)),
    ]
    if s == 0:
        return pl.pallas_call(_tc_body_first, in_specs=data_specs, **common)(
            patches, W_obs, b_obs, act_s
        )
    return pl.pallas_call(
        _tc_body_chain,
        in_specs=[pl.BlockSpec(memory_space=pltpu.MemorySpace.HBM)] + data_specs,
        input_output_aliases={0: 0},
        **common,
    )(prev_out, patches, W_obs, b_obs, act_s)


def kernel(patches, action, W_obs, b_obs, emb_table):
    offsets = (jnp.arange(NUM_ACTIONS, dtype=action.dtype) * ACTION_DIM)[None, :]
    idx = (action + offsets).reshape(-1)
    acts = []
    for s in range(NSPLIT):
        idx_s = lax.slice(idx, (s * ROWS_H,), ((s + 1) * ROWS_H,))
        act_s = _sc_gather()(idx_s, emb_table)
        acts.append(act_s.reshape(BH, NUM_ACTIONS, EMB_DIM))
    out = None
    b2 = b_obs.reshape(1, EMB_DIM)
    for s in range(NSPLIT):
        out = _tc_call(s, out, patches, W_obs, b2, acts[s])
    return out


# speedup vs baseline: 1.2587x; 1.2587x over previous
import functools

import jax
import jax.numpy as jnp
from jax import lax
from jax.experimental import pallas as pl
from jax.experimental.pallas import tpu as pltpu
from jax.experimental.pallas import tpu_sc as plsc

NUM_ACTIONS = 100
ACTION_DIM = 1000
PATCHDES_DIM = 256
EMB_DIM = 128
BATCH = 1024
NUM_PATCHES = 196
ACT_VOCAB = NUM_ACTIONS * ACTION_DIM
SEQ = NUM_PATCHES + NUM_ACTIONS

NTOT = BATCH * NUM_ACTIONS
NW = 32
PER_W = NTOT // NW
CHUNK = 128
NCHUNK = PER_W // CHUNK


def _sc_body(idx_hbm, dst_hbm, table_hbm, out_hbm, idx_v, dst_v, rows_v, sg, sw):
    wid = lax.axis_index("s") * 2 + lax.axis_index("c")
    base = wid * PER_W
    pltpu.sync_copy(idx_hbm.at[pl.ds(base, PER_W)], idx_v)
    pltpu.sync_copy(dst_hbm.at[wid], dst_v)

    def step(j, carry):
        off = pl.multiple_of(j * CHUNK, CHUNK)
        pltpu.async_copy(
            table_hbm.at[idx_v.at[pl.ds(off, CHUNK)]], rows_v, sg
        ).wait()
        pltpu.async_copy(rows_v, out_hbm.at[dst_v.at[j]], sw).wait()
        return carry

    lax.fori_loop(0, NCHUNK, step, 0)


@functools.lru_cache(maxsize=1)
def _sc_scatter():
    return pl.kernel(
        _sc_body,
        out_type=jax.ShapeDtypeStruct((BATCH * SEQ, EMB_DIM), jnp.float32),
        mesh=plsc.VectorSubcoreMesh(core_axis_name="c", subcore_axis_name="s"),
        scratch_types=[
            pltpu.VMEM((PER_W,), jnp.int32),
            pltpu.VMEM((NCHUNK, CHUNK), jnp.int32),
            pltpu.VMEM((CHUNK, EMB_DIM), jnp.float32),
            pltpu.SemaphoreType.DMA,
            pltpu.SemaphoreType.DMA,
        ],
    )


TC_ROWS = 200


def _tc_body(a_ref, p_ref, w_ref, b_ref, ah_ref, o_ref):
    del a_ref
    x = p_ref[...].reshape(-1, PATCHDES_DIM)
    y = jnp.dot(x, w_ref[...], preferred_element_type=jnp.float32) + b_ref[...]
    o_ref[:, :NUM_PATCHES, :] = y.reshape(-1, NUM_PATCHES, EMB_DIM)
    o_ref[:, NUM_PATCHES:, :] = ah_ref[...]


def _tc_call(partial_out, patches, W_obs, b_obs, act_head, bsz=64):
    grid = BATCH // bsz
    return pl.pallas_call(
        _tc_body,
        grid=(grid,),
        in_specs=[
            pl.BlockSpec(memory_space=pltpu.MemorySpace.HBM),
            pl.BlockSpec((bsz, NUM_PATCHES, PATCHDES_DIM), lambda i: (i, 0, 0)),
            pl.BlockSpec((PATCHDES_DIM, EMB_DIM), lambda i: (0, 0)),
            pl.BlockSpec((1, EMB_DIM), lambda i: (0, 0)),
            pl.BlockSpec((bsz, TC_ROWS - NUM_PATCHES, EMB_DIM), lambda i: (i, 0, 0)),
        ],
        out_specs=pl.BlockSpec((bsz, TC_ROWS, EMB_DIM), lambda i: (i, 0, 0)),
        out_shape=jax.ShapeDtypeStruct((BATCH, SEQ, EMB_DIM), jnp.float32),
        input_output_aliases={0: 0},
        compiler_params=pltpu.CompilerParams(
            dimension_semantics=("arbitrary",),
        ),
    )(partial_out, patches, W_obs, b_obs, act_head)


def kernel(patches, action, W_obs, b_obs, emb_table):
    offsets = (jnp.arange(NUM_ACTIONS, dtype=action.dtype) * ACTION_DIM)[None, :]
    idx = (action + offsets).reshape(-1)
    p = jnp.arange(NTOT, dtype=jnp.int32)
    dst = (p // NUM_ACTIONS) * SEQ + NUM_PATCHES + (p % NUM_ACTIONS)
    dst3 = dst.reshape(NW, NCHUNK, CHUNK)
    partial_out = _sc_scatter()(idx, dst3, emb_table)
    partial_out = partial_out.reshape(BATCH, SEQ, EMB_DIM)
    act_head = lax.slice(
        partial_out, (0, NUM_PATCHES, 0), (BATCH, TC_ROWS, EMB_DIM)
    )
    return _tc_call(
        partial_out, patches, W_obs, b_obs.reshape(1, EMB_DIM), act_head
    )


# speedup vs baseline: 1.2592x; 1.0004x over previous
import functools

import jax
import jax.numpy as jnp
from jax import lax
from jax.experimental import pallas as pl
from jax.experimental.pallas import tpu as pltpu
from jax.experimental.pallas import tpu_sc as plsc

NUM_ACTIONS = 100
ACTION_DIM = 1000
PATCHDES_DIM = 256
EMB_DIM = 128
BATCH = 1024
NUM_PATCHES = 196
ACT_VOCAB = NUM_ACTIONS * ACTION_DIM
SEQ = NUM_PATCHES + NUM_ACTIONS

NTOT = BATCH * NUM_ACTIONS
NW = 32
PER_W = NTOT // NW
CHUNK = 128
NCHUNK = PER_W // CHUNK


def _sc_body(idx_hbm, dst_hbm, table_hbm, out_hbm, idx_v, dst_v, rows_v, sg, sw):
    wid = lax.axis_index("s") * 2 + lax.axis_index("c")
    base = wid * PER_W
    pltpu.sync_copy(idx_hbm.at[pl.ds(base, PER_W)], idx_v)
    pltpu.sync_copy(dst_hbm.at[wid], dst_v)

    def step(j, carry):
        off = pl.multiple_of(j * CHUNK, CHUNK)
        pltpu.async_copy(
            table_hbm.at[idx_v.at[pl.ds(off, CHUNK)]], rows_v, sg
        ).wait()
        pltpu.async_copy(rows_v, out_hbm.at[dst_v.at[j]], sw).wait()
        return carry

    lax.fori_loop(0, NCHUNK, step, 0)


@functools.lru_cache(maxsize=1)
def _sc_scatter():
    return pl.kernel(
        _sc_body,
        out_type=jax.ShapeDtypeStruct((BATCH * SEQ, EMB_DIM), jnp.float32),
        mesh=plsc.VectorSubcoreMesh(core_axis_name="c", subcore_axis_name="s"),
        scratch_types=[
            pltpu.VMEM((PER_W,), jnp.int32),
            pltpu.VMEM((NCHUNK, CHUNK), jnp.int32),
            pltpu.VMEM((CHUNK, EMB_DIM), jnp.float32),
            pltpu.SemaphoreType.DMA,
            pltpu.SemaphoreType.DMA,
        ],
    )


TC_ROWS = 200


def _tc_body(a_ref, p_ref, w_ref, b_ref, ah_ref, o_ref):
    del a_ref
    x = p_ref[...].reshape(-1, PATCHDES_DIM)
    y = jnp.dot(x, w_ref[...], preferred_element_type=jnp.float32) + b_ref[...]
    o_ref[:, :NUM_PATCHES, :] = y.reshape(-1, NUM_PATCHES, EMB_DIM)
    o_ref[:, NUM_PATCHES:, :] = ah_ref[...]


def _tc_call(partial_out, patches, W_obs, b_obs, act_head, bsz=64):
    grid = BATCH // bsz
    return pl.pallas_call(
        _tc_body,
        grid=(grid,),
        in_specs=[
            pl.BlockSpec(memory_space=pltpu.MemorySpace.HBM),
            pl.BlockSpec((bsz, NUM_PATCHES, PATCHDES_DIM), lambda i: (i, 0, 0)),
            pl.BlockSpec((PATCHDES_DIM, EMB_DIM), lambda i: (0, 0)),
            pl.BlockSpec((1, EMB_DIM), lambda i: (0, 0)),
            pl.BlockSpec((bsz, TC_ROWS - NUM_PATCHES, EMB_DIM), lambda i: (i, 0, 0)),
        ],
        out_specs=pl.BlockSpec((bsz, TC_ROWS, EMB_DIM), lambda i: (i, 0, 0)),
        out_shape=jax.ShapeDtypeStruct((BATCH, SEQ, EMB_DIM), jnp.float32),
        input_output_aliases={0: 0},
        compiler_params=pltpu.CompilerParams(
            dimension_semantics=("parallel",),
        ),
    )(partial_out, patches, W_obs, b_obs, act_head)


def kernel(patches, action, W_obs, b_obs, emb_table):
    offsets = (jnp.arange(NUM_ACTIONS, dtype=action.dtype) * ACTION_DIM)[None, :]
    idx = (action + offsets).reshape(-1)
    p = jnp.arange(NTOT, dtype=jnp.int32)
    dst = (p // NUM_ACTIONS) * SEQ + NUM_PATCHES + (p % NUM_ACTIONS)
    dst3 = dst.reshape(NW, NCHUNK, CHUNK)
    partial_out = _sc_scatter()(idx, dst3, emb_table)
    partial_out = partial_out.reshape(BATCH, SEQ, EMB_DIM)
    act_head = lax.slice(
        partial_out, (0, NUM_PATCHES, 0), (BATCH, TC_ROWS, EMB_DIM)
    )
    return _tc_call(
        partial_out, patches, W_obs, b_obs.reshape(1, EMB_DIM), act_head
    )
